# Initial kernel scaffold; baseline (speedup 1.0000x reference)
import jax
import jax.numpy as jnp
from jax.experimental import pallas as pl


def kernel(prediction, target, weights, thresholds):
    raise NotImplementedError("write your pallas kernel here")


# speedup vs baseline: 11254.4186x; 11254.4186x over previous
import functools

import jax
import jax.numpy as jnp
from jax.experimental import pallas as pl
from jax.experimental.pallas import tpu as pltpu


def _wmse_body(pred_ref, tgt_ref, w_ref, t_ref, out_ref, acc_ref):
    i = pl.program_id(0)
    n = pl.num_programs(0)

    t = tgt_ref[...]
    p = pred_ref[...]
    d = p - t
    sq = d * d
    w = jnp.full_like(t, w_ref[0])
    for k in range(4):
        w = jnp.where(t >= t_ref[k], w_ref[k + 1], w)
    partial = jnp.sum(w * sq, axis=0)

    @pl.when(i == 0)
    def _init():
        acc_ref[...] = jnp.zeros_like(acc_ref)

    acc_ref[...] += partial.reshape(acc_ref.shape)

    @pl.when(i == n - 1)
    def _fin():
        out_ref[0] = jnp.sum(acc_ref[...])


def kernel(prediction, target, weights, thresholds):
    total = prediction.size
    rows = total // 1024
    p2 = prediction.reshape(rows, 1024)
    t2 = target.reshape(rows, 1024)

    block_rows = 512
    grid = rows // block_rows

    out = pl.pallas_call(
        _wmse_body,
        grid=(grid,),
        in_specs=[
            pl.BlockSpec((block_rows, 1024), lambda i: (i, 0)),
            pl.BlockSpec((block_rows, 1024), lambda i: (i, 0)),
            pl.BlockSpec(memory_space=pltpu.SMEM),
            pl.BlockSpec(memory_space=pltpu.SMEM),
        ],
        out_specs=pl.BlockSpec(memory_space=pltpu.SMEM),
        out_shape=jax.ShapeDtypeStruct((1,), jnp.float32),
        scratch_shapes=[pltpu.VMEM((8, 128), jnp.float32)],
    )(p2, t2, weights, thresholds)
    return (out[0] / total).astype(jnp.float32).reshape(())
